# baseline (speedup 1.0000x reference)
import jax
import jax.numpy as jnp
from jax.experimental import pallas as pl
from jax.experimental.pallas import tpu as pltpu

_EPS = 1e-6

_TB = 32
_BH = 16


def _cell_kernel(x_ref, wk_ref, wv_ref, wq_ref, wkr_ref, u0_ref, v_ref,
                 out_ref, uf_ref, k_s, v_s, q_s, u_s):
    tb = pl.program_id(1)
    nt = pl.num_programs(1)
    n = wk_ref.shape[1]

    @pl.when(tb == 0)
    def _():
        u_s[...] = u0_ref[...]

    xb = x_ref[...].reshape(_TB * _BH, x_ref.shape[2])
    k = jnp.dot(xb, wk_ref[...], preferred_element_type=jnp.float32)
    ss = jnp.sum(k * k, axis=-1, keepdims=True)
    inv = 1.0 / (jnp.sqrt(ss) + _EPS)
    k_s[...] = k * inv
    v_s[...] = jnp.dot(xb, wv_ref[...], preferred_element_type=jnp.float32)
    q_s[...] = jnp.dot(xb, wq_ref[...], preferred_element_type=jnp.float32)

    nc = 256

    def step(t, carry):
        sl = pl.ds(pl.multiple_of(t * _BH, _BH), _BH)
        rr = v_ref.shape[0]
        vtk = jnp.zeros((rr, _BH, 1), jnp.float32)
        krt = jnp.zeros((rr, _BH, 1), jnp.float32)
        vtq = jnp.zeros((rr, _BH, 1), jnp.float32)
        for c in range(0, n, nc):
            cs = slice(c, c + nc)
            knc = k_s[sl, cs][None]
            qtc = q_s[sl, cs][None]
            v4c = v_ref[:, :, cs]
            wkrc = wkr_ref[:, :, cs]
            vtk = vtk + jnp.sum(v4c * knc, axis=-1, keepdims=True)
            krt = krt + jnp.sum(wkrc * knc, axis=-1, keepdims=True)
            vtq = vtq + jnp.sum(v4c * qtc, axis=-1, keepdims=True)
        for c in range(0, n, nc):
            cs = slice(c, c + nc)
            u = u_s[:, :, cs]
            retr = jnp.sum(u * vtk, axis=0)
            delta = v_s[sl, cs] - retr
            un = jnp.tanh(u + delta[None] * krt)
            u_s[:, :, cs] = un
            sq = jnp.sum(un * vtq, axis=0)
            sg = 1.0 / (1.0 + jnp.exp(-sq))
            out_ref[t, :, cs] = sq * (sq * sg)
        return carry

    jax.lax.fori_loop(0, _TB, step, 0)

    @pl.when(tb == nt - 1)
    def _():
        uf_ref[...] = u_s[...]


def kernel(x, W_k, W_v, W_q, W_kr, U0, V0):
    T, B, D = x.shape
    n = W_k.shape[0]
    r = W_kr.shape[0]
    nt = T // _TB

    wkT = W_k.T
    wvT = W_v.T
    wqT = W_q.T
    wkr_b = jnp.broadcast_to(W_kr[:, None, :], (r, _BH, n))
    u0t = U0.transpose(2, 0, 1)
    vt = V0.transpose(2, 0, 1)

    out, uf = pl.pallas_call(
        _cell_kernel,
        grid=(B // _BH, nt),
        in_specs=[
            pl.BlockSpec((_TB, _BH, D), lambda i, t: (t, i, 0)),
            pl.BlockSpec((D, n), lambda i, t: (0, 0)),
            pl.BlockSpec((D, n), lambda i, t: (0, 0)),
            pl.BlockSpec((D, n), lambda i, t: (0, 0)),
            pl.BlockSpec((r, _BH, n), lambda i, t: (0, 0, 0)),
            pl.BlockSpec((r, _BH, n), lambda i, t: (0, i, 0)),
            pl.BlockSpec((r, _BH, n), lambda i, t: (0, i, 0)),
        ],
        out_specs=[
            pl.BlockSpec((_TB, _BH, n), lambda i, t: (t, i, 0)),
            pl.BlockSpec((r, _BH, n), lambda i, t: (0, i, 0)),
        ],
        out_shape=[
            jax.ShapeDtypeStruct((T, B, n), jnp.float32),
            jax.ShapeDtypeStruct((r, B, n), jnp.float32),
        ],
        scratch_shapes=[
            pltpu.VMEM((_TB * _BH, n), jnp.float32),
            pltpu.VMEM((_TB * _BH, n), jnp.float32),
            pltpu.VMEM((_TB * _BH, n), jnp.float32),
            pltpu.VMEM((r, _BH, n), jnp.float32),
        ],
        compiler_params=pltpu.CompilerParams(
            dimension_semantics=("parallel", "arbitrary"),
            vmem_limit_bytes=50 * 1024 * 1024,
        ),
        name="e74_low_rank_cell",
    )(x, wkT, wvT, wqT, wkr_b, u0t, vt)

    return out, (uf.transpose(1, 2, 0), V0)


# speedup vs baseline: 1.0719x; 1.0719x over previous
import jax
import jax.numpy as jnp
from jax.experimental import pallas as pl
from jax.experimental.pallas import tpu as pltpu

_EPS = 1e-6

_TB = 32
_BH = 16


def _cell_kernel(x_ref, wk_ref, wv_ref, wq_ref, wkr_ref, u0_ref, v_ref,
                 out_ref, uf_ref, k_s, v_s, q_s, u_s):
    tb = pl.program_id(1)
    nt = pl.num_programs(1)
    n = wk_ref.shape[1]

    @pl.when(tb == 0)
    def _():
        u_s[...] = u0_ref[...]

    xb = x_ref[...].reshape(_TB * _BH, x_ref.shape[2])
    k = jnp.dot(xb, wk_ref[...], preferred_element_type=jnp.float32)
    ss = jnp.sum(k * k, axis=-1, keepdims=True)
    inv = 1.0 / (jnp.sqrt(ss) + _EPS)
    k_s[...] = k * inv
    v_s[...] = jnp.dot(xb, wv_ref[...], preferred_element_type=jnp.float32)
    q_s[...] = jnp.dot(xb, wq_ref[...], preferred_element_type=jnp.float32)

    nc = 256

    def step(t, carry):
        sl = pl.ds(pl.multiple_of(t * _BH, _BH), _BH)
        rr = v_ref.shape[0]
        vtk = jnp.zeros((rr, _BH, 1), jnp.float32)
        krt = jnp.zeros((rr, _BH, 1), jnp.float32)
        vtq = jnp.zeros((rr, _BH, 1), jnp.float32)
        for c in range(0, n, nc):
            cs = slice(c, c + nc)
            knc = k_s[sl, cs][None]
            qtc = q_s[sl, cs][None]
            v4c = v_ref[:, :, cs]
            wkrc = wkr_ref[:, :, cs]
            vtk = vtk + jnp.sum(v4c * knc, axis=-1, keepdims=True)
            krt = krt + jnp.sum(wkrc * knc, axis=-1, keepdims=True)
            vtq = vtq + jnp.sum(v4c * qtc, axis=-1, keepdims=True)
        for c in range(0, n, nc):
            cs = slice(c, c + nc)
            u = u_s[:, :, cs]
            retr = jnp.sum(u * vtk, axis=0)
            delta = v_s[sl, cs] - retr
            un = jnp.tanh(u + delta[None] * krt)
            u_s[:, :, cs] = un
            sq = jnp.sum(un * vtq, axis=0)
            sg = 1.0 / (1.0 + jnp.exp(-sq))
            out_ref[t, :, cs] = sq * (sq * sg)
        return carry

    jax.lax.fori_loop(0, _TB, step, 0, unroll=2)

    @pl.when(tb == nt - 1)
    def _():
        uf_ref[...] = u_s[...]


def kernel(x, W_k, W_v, W_q, W_kr, U0, V0):
    T, B, D = x.shape
    n = W_k.shape[0]
    r = W_kr.shape[0]
    nt = T // _TB

    wkT = W_k.T
    wvT = W_v.T
    wqT = W_q.T
    wkr_b = jnp.broadcast_to(W_kr[:, None, :], (r, _BH, n))
    u0t = U0.transpose(2, 0, 1)
    vt = V0.transpose(2, 0, 1)

    out, uf = pl.pallas_call(
        _cell_kernel,
        grid=(B // _BH, nt),
        in_specs=[
            pl.BlockSpec((_TB, _BH, D), lambda i, t: (t, i, 0)),
            pl.BlockSpec((D, n), lambda i, t: (0, 0)),
            pl.BlockSpec((D, n), lambda i, t: (0, 0)),
            pl.BlockSpec((D, n), lambda i, t: (0, 0)),
            pl.BlockSpec((r, _BH, n), lambda i, t: (0, 0, 0)),
            pl.BlockSpec((r, _BH, n), lambda i, t: (0, i, 0)),
            pl.BlockSpec((r, _BH, n), lambda i, t: (0, i, 0)),
        ],
        out_specs=[
            pl.BlockSpec((_TB, _BH, n), lambda i, t: (t, i, 0)),
            pl.BlockSpec((r, _BH, n), lambda i, t: (0, i, 0)),
        ],
        out_shape=[
            jax.ShapeDtypeStruct((T, B, n), jnp.float32),
            jax.ShapeDtypeStruct((r, B, n), jnp.float32),
        ],
        scratch_shapes=[
            pltpu.VMEM((_TB * _BH, n), jnp.float32),
            pltpu.VMEM((_TB * _BH, n), jnp.float32),
            pltpu.VMEM((_TB * _BH, n), jnp.float32),
            pltpu.VMEM((r, _BH, n), jnp.float32),
        ],
        compiler_params=pltpu.CompilerParams(
            dimension_semantics=("parallel", "arbitrary"),
            vmem_limit_bytes=50 * 1024 * 1024,
        ),
        name="e74_low_rank_cell",
    )(x, wkT, wvT, wqT, wkr_b, u0t, vt)

    return out, (uf.transpose(1, 2, 0), V0)


# speedup vs baseline: 1.3574x; 1.2664x over previous
import jax
import jax.numpy as jnp
from jax.experimental import pallas as pl
from jax.experimental.pallas import tpu as pltpu

_EPS = 1e-6

_TB = 32
_BH = 16


def _cell_kernel(x_ref, wk_ref, wv_ref, wq_ref, wkr_ref, u0_ref, v_ref,
                 out_ref, uf_ref, k_s, v_s, q_s, u_s):
    tb = pl.program_id(1)
    nt = pl.num_programs(1)
    n = wk_ref.shape[1]

    @pl.when(tb == 0)
    def _():
        u_s[...] = u0_ref[...]

    xb = x_ref[...].reshape(_TB * _BH, x_ref.shape[2])
    k = jnp.dot(xb, wk_ref[...], preferred_element_type=jnp.float32)
    ss = jnp.sum(k * k, axis=-1, keepdims=True)
    inv = 1.0 / (jnp.sqrt(ss) + _EPS)
    k_s[...] = k * inv
    v_s[...] = jnp.dot(xb, wv_ref[...], preferred_element_type=jnp.float32)
    q_s[...] = jnp.dot(xb, wq_ref[...], preferred_element_type=jnp.float32)

    v4 = v_ref[...]
    wkr4 = wkr_ref[...]

    def step(t, carry):
        sl = pl.ds(pl.multiple_of(t * _BH, _BH), _BH)
        kn = k_s[sl, :]
        vt = v_s[sl, :]
        qt = q_s[sl, :]
        kn3 = kn[None]
        u = u_s[...]
        vtk = jnp.sum(v4 * kn3, axis=-1, keepdims=True)
        retr = jnp.sum(u * vtk, axis=0)
        delta = vt - retr
        krt = jnp.sum(wkr4 * kn3, axis=-1, keepdims=True)
        un = jnp.tanh(u + delta[None] * krt)
        u_s[...] = un
        vtq = jnp.sum(v4 * qt[None], axis=-1, keepdims=True)
        sq = jnp.sum(un * vtq, axis=0)
        sg = 1.0 / (1.0 + jnp.exp(-sq))
        out_ref[t] = sq * (sq * sg)
        return carry

    jax.lax.fori_loop(0, _TB, step, 0, unroll=4)

    @pl.when(tb == nt - 1)
    def _():
        uf_ref[...] = u_s[...]


def kernel(x, W_k, W_v, W_q, W_kr, U0, V0):
    T, B, D = x.shape
    n = W_k.shape[0]
    r = W_kr.shape[0]
    nt = T // _TB

    wkT = W_k.T
    wvT = W_v.T
    wqT = W_q.T
    wkr_b = jnp.broadcast_to(W_kr[:, None, :], (r, _BH, n))
    u0t = U0.transpose(2, 0, 1)
    vt = V0.transpose(2, 0, 1)

    out, uf = pl.pallas_call(
        _cell_kernel,
        grid=(B // _BH, nt),
        in_specs=[
            pl.BlockSpec((_TB, _BH, D), lambda i, t: (t, i, 0)),
            pl.BlockSpec((D, n), lambda i, t: (0, 0)),
            pl.BlockSpec((D, n), lambda i, t: (0, 0)),
            pl.BlockSpec((D, n), lambda i, t: (0, 0)),
            pl.BlockSpec((r, _BH, n), lambda i, t: (0, 0, 0)),
            pl.BlockSpec((r, _BH, n), lambda i, t: (0, i, 0)),
            pl.BlockSpec((r, _BH, n), lambda i, t: (0, i, 0)),
        ],
        out_specs=[
            pl.BlockSpec((_TB, _BH, n), lambda i, t: (t, i, 0)),
            pl.BlockSpec((r, _BH, n), lambda i, t: (0, i, 0)),
        ],
        out_shape=[
            jax.ShapeDtypeStruct((T, B, n), jnp.float32),
            jax.ShapeDtypeStruct((r, B, n), jnp.float32),
        ],
        scratch_shapes=[
            pltpu.VMEM((_TB * _BH, n), jnp.float32),
            pltpu.VMEM((_TB * _BH, n), jnp.float32),
            pltpu.VMEM((_TB * _BH, n), jnp.float32),
            pltpu.VMEM((r, _BH, n), jnp.float32),
        ],
        compiler_params=pltpu.CompilerParams(
            dimension_semantics=("parallel", "arbitrary"),
            vmem_limit_bytes=50 * 1024 * 1024,
        ),
        name="e74_low_rank_cell",
    )(x, wkT, wvT, wqT, wkr_b, u0t, vt)

    return out, (uf.transpose(1, 2, 0), V0)
